# baseline (speedup 1.0000x reference)
import functools
import math

import jax
import jax.numpy as jnp
from jax import lax
from jax.experimental import pallas as pl
from jax.experimental.pallas import tpu as pltpu
from jax.experimental.pallas import tpu_sc as plsc

F32 = jnp.float32

NC = 2
NS = 16
NW = NC * NS
LANES = 16

CHUNK = 128
F = 64
BN = 512


def _cdiv(a, b):
    return (a + b - 1) // b


def _make_deg_kernel(n_pad, chunks):
    rows_per_tile = n_pad // NS
    zsteps = rows_per_tile // CHUNK
    mesh = plsc.VectorSubcoreMesh(core_axis_name="c", subcore_axis_name="s")

    @functools.partial(
        pl.kernel,
        out_type=jax.ShapeDtypeStruct((NC * 3 * n_pad,), F32),
        mesh=mesh,
        scratch_types=[
            pltpu.VMEM((chunks, CHUNK), jnp.int32),
            pltpu.VMEM((chunks, CHUNK), F32),
            pltpu.VMEM((rows_per_tile,), F32),
            pltpu.VMEM_SHARED((n_pad,), F32),
            pltpu.VMEM_SHARED((n_pad,), F32),
            pltpu.VMEM_SHARED((n_pad,), F32),
        ],
    )
    def deg_kernel(src_hbm, ew_hbm, zrow_hbm, out_hbm,
                   src_v, ew_v, stage_v, acc0, acc1, acc2):
        c = lax.axis_index("c")
        s = lax.axis_index("s")
        wid = c * NS + s
        accs = [acc0, acc1, acc2]
        pltpu.sync_copy(zrow_hbm, stage_v)
        for acc in accs:
            pltpu.sync_copy(stage_v, acc.at[pl.ds(s * rows_per_tile, rows_per_tile)])
        plsc.subcore_barrier()
        for k, acc in enumerate(accs):
            pltpu.sync_copy(src_hbm.at[k, wid], src_v)
            pltpu.sync_copy(ew_hbm.at[k, wid], ew_v)

            def body(j, _, acc=acc):
                pltpu.sync_copy(ew_v.at[j], acc.at[src_v.at[j]], add=True)
                return 0

            lax.fori_loop(0, chunks, body, 0)
        plsc.subcore_barrier()
        for k, acc in enumerate(accs):
            pltpu.sync_copy(acc.at[pl.ds(s * rows_per_tile, rows_per_tile)], stage_v)
            off = (c * 3 + k) * n_pad + s * rows_per_tile
            pltpu.sync_copy(stage_v, out_hbm.at[pl.ds(off, rows_per_tile)])

    return deg_kernel


NBUF = 2


def _make_conv_kernel(n_pad, chunks, ch_slab, weighted):
    rows_per_tile = n_pad // NS
    zsteps = rows_per_tile // CHUNK
    W = 2 * F
    mesh = plsc.VectorSubcoreMesh(core_axis_name="c", subcore_axis_name="s")

    @functools.partial(
        pl.kernel,
        out_type=jax.ShapeDtypeStruct((NC, n_pad, W), F32),
        mesh=mesh,
        scratch_types=[
            pltpu.VMEM((ch_slab, CHUNK), jnp.int32),
            pltpu.VMEM((ch_slab, CHUNK), jnp.int32),
            pltpu.VMEM((LANES, CHUNK), F32),
            pltpu.VMEM((CHUNK, W), F32),
            pltpu.VMEM_SHARED((n_pad, W), F32),
        ],
    )
    def conv_kernel(h_hbm, src_hbm, dst_hbm, ew_hbm, zblk_hbm, out_hbm,
                    src_v, dst_v, ew_v, rows_v, acc):
        c = lax.axis_index("c")
        s = lax.axis_index("s")
        wid = c * NS + s
        pltpu.sync_copy(zblk_hbm, rows_v)
        for z in range(zsteps):
            off = s * rows_per_tile + z * CHUNK
            pltpu.sync_copy(rows_v, acc.at[pl.ds(off, CHUNK)])
        plsc.subcore_barrier()
        pltpu.sync_copy(src_hbm.at[wid], src_v)
        pltpu.sync_copy(dst_hbm.at[wid], dst_v)

        def chunk_body(j, _):
            pltpu.sync_copy(h_hbm.at[src_v.at[j]], rows_v)
            if weighted:
                pltpu.sync_copy(ew_hbm.at[wid, j], ew_v)

                def grp(g, _g):
                    for l in range(LANES):
                        i = g * LANES + l
                        bv = ew_v[2 * g + l // 8, pl.ds((l % 8) * LANES, LANES)]
                        for q in range(F // LANES):
                            rows_v[i, pl.ds(q * LANES, LANES)] = (
                                rows_v[i, pl.ds(q * LANES, LANES)] * bv)
                    return 0

                lax.fori_loop(0, CHUNK // LANES, grp, 0)
            pltpu.sync_copy(rows_v, acc.at[dst_v.at[j]], add=True)
            return 0

        lax.fori_loop(0, chunks, chunk_body, 0)
        plsc.subcore_barrier()
        for z in range(zsteps):
            off = s * rows_per_tile + z * CHUNK
            pltpu.sync_copy(acc.at[pl.ds(off, CHUNK)], rows_v)
            pltpu.sync_copy(rows_v, out_hbm.at[c, pl.ds(off, CHUNK)])

    return conv_kernel


def _make_block_kernel(n_pad, chunks, ch_slab):
    rows_per_tile = n_pad // NS
    zsteps = rows_per_tile // CHUNK
    W = 2 * F
    mesh = plsc.VectorSubcoreMesh(core_axis_name="c", subcore_axis_name="s")

    @functools.partial(
        pl.kernel,
        out_type=[jax.ShapeDtypeStruct((NC, n_pad, W), F32)] * 3,
        mesh=mesh,
        scratch_types=[
            pltpu.VMEM((NBUF, CHUNK), jnp.int32),
            pltpu.VMEM((NBUF, CHUNK), jnp.int32),
            pltpu.VMEM((NBUF, LANES, CHUNK), F32),
            pltpu.VMEM((NBUF, CHUNK, W), F32),
            pltpu.VMEM_SHARED((n_pad, W), F32),
        ] + [pltpu.SemaphoreType.DMA] * (4 * NBUF),
    )
    def block_kernel(h0_hbm, h1_hbm, h2_hbm, s0_hbm, s1_hbm, s2_hbm,
                     d0_hbm, d1_hbm, d2_hbm, ew1_hbm, ew2_hbm, zblk_hbm,
                     r0_hbm, r1_hbm, r2_hbm,
                     src_v, dst_v, ew_v, rows_v, acc, *sems):
        gsem = sems[0:NBUF]
        esem = sems[NBUF:2 * NBUF]
        isem = sems[2 * NBUF:3 * NBUF]
        dsem = sems[3 * NBUF:4 * NBUF]
        c = lax.axis_index("c")
        s = lax.axis_index("s")
        wid = c * NS + s

        def run_set(h_hbm, src_hbm, dst_hbm, ew_hbm, out_hbm, weighted):
            base = wid * (ch_slab * CHUNK)

            def idx_slice(j):
                return pl.ds(base + j * CHUNK, CHUNK)

            for b in range(NBUF):
                pltpu.sync_copy(zblk_hbm, rows_v.at[b])
            for z in range(zsteps):
                off = s * rows_per_tile + z * CHUNK
                pltpu.sync_copy(rows_v.at[0], acc.at[pl.ds(off, CHUNK)])
            plsc.subcore_barrier()
            pltpu.sync_copy(src_hbm.at[idx_slice(0)], src_v.at[0])
            pltpu.async_copy(dst_hbm.at[idx_slice(0)], dst_v.at[0], dsem[0])
            pltpu.async_copy(h_hbm.at[src_v.at[0]], rows_v.at[0], gsem[0])
            pltpu.async_copy(src_hbm.at[idx_slice(1)], src_v.at[1], isem[1])
            if weighted:
                pltpu.async_copy(ew_hbm.at[wid, 0], ew_v.at[0], esem[0])
                pltpu.async_copy(ew_hbm.at[wid, 1], ew_v.at[1], esem[1])

            def turn(j, b):
                o = 1 - b
                pltpu.async_copy(dst_hbm.at[idx_slice(j + 1)], dst_v.at[o],
                                 dsem[o])
                pltpu.make_async_copy(src_hbm.at[idx_slice(0)], src_v.at[o],
                                      isem[o]).wait()
                pltpu.async_copy(h_hbm.at[src_v.at[o]], rows_v.at[o],
                                 gsem[o])
                pltpu.make_async_copy(h_hbm.at[src_v.at[0]], rows_v.at[b],
                                      gsem[b]).wait()
                pltpu.async_copy(src_hbm.at[idx_slice(j + 2)], src_v.at[b],
                                 isem[b])
                if weighted:
                    pltpu.make_async_copy(ew_hbm.at[wid, 0], ew_v.at[b],
                                          esem[b]).wait()

                    def grp(g, _):
                        for l in range(LANES):
                            i = g * LANES + l
                            bv = ew_v[b, 2 * g + l // 8,
                                      pl.ds((l % 8) * LANES, LANES)]
                            for q in range(F // LANES):
                                rows_v[b, i, pl.ds(q * LANES, LANES)] = (
                                    rows_v[b, i, pl.ds(q * LANES, LANES)] * bv)
                        return 0

                    lax.fori_loop(0, CHUNK // LANES, grp, 0)
                    pltpu.async_copy(ew_hbm.at[wid, j + 2], ew_v.at[b],
                                     esem[b])
                pltpu.make_async_copy(dst_hbm.at[idx_slice(0)], dst_v.at[b],
                                      dsem[b]).wait()
                pltpu.sync_copy(rows_v.at[b], acc.at[dst_v.at[b]], add=True)

            def body(r, _):
                j0 = r * NBUF
                for b in range(NBUF):
                    turn(j0 + b, b)
                return 0

            lax.fori_loop(0, chunks // NBUF, body, 0)
            pltpu.make_async_copy(h_hbm.at[src_v.at[0]],
                                  rows_v.at[chunks % NBUF],
                                  gsem[chunks % NBUF]).wait()
            pltpu.make_async_copy(src_hbm.at[idx_slice(0)],
                                  src_v.at[(chunks + 1) % NBUF],
                                  isem[(chunks + 1) % NBUF]).wait()
            pltpu.make_async_copy(dst_hbm.at[idx_slice(0)],
                                  dst_v.at[chunks % NBUF],
                                  dsem[chunks % NBUF]).wait()
            if weighted:
                for b in range(NBUF):
                    pltpu.make_async_copy(ew_hbm.at[wid, 0], ew_v.at[b],
                                          esem[b]).wait()
            plsc.subcore_barrier()
            for z in range(zsteps):
                off = s * rows_per_tile + z * CHUNK
                pltpu.sync_copy(acc.at[pl.ds(off, CHUNK)], rows_v.at[0])
                pltpu.sync_copy(rows_v.at[0], out_hbm.at[c, pl.ds(off, CHUNK)])

        run_set(h0_hbm, s0_hbm, d0_hbm, None, r0_hbm, False)
        run_set(h1_hbm, s1_hbm, d1_hbm, ew1_hbm, r1_hbm, True)
        run_set(h2_hbm, s2_hbm, d2_hbm, ew2_hbm, r2_hbm, True)

    return block_kernel


def _dis_from_deg(deg_blk):
    d = deg_blk[0] + deg_blk[1]
    return jnp.where(d > 0, lax.rsqrt(d), jnp.zeros_like(d))


def _tc1_body(x_ref, w_ref, deg_ref, h0_ref, h1_ref, h2_ref):
    h = lax.dot_general(x_ref[...], w_ref[...], (((1,), (1,)), ((), ())),
                        preferred_element_type=F32)
    dis = _dis_from_deg(deg_ref[...])
    z = jnp.zeros_like(h)
    h0_ref[...] = jnp.concatenate([h * dis[0][:, None], z], axis=1)
    h1_ref[...] = jnp.concatenate([h * dis[1][:, None], z], axis=1)
    h2_ref[...] = jnp.concatenate([h * dis[2][:, None], z], axis=1)


def _tc_mid_body(r0_ref, r1_ref, r2_ref, deg_ref, b_ref, w_ref,
                 o0_ref, o1_ref, o2_ref):
    dis = _dis_from_deg(deg_ref[...])
    ys = []
    for k, r_ref in enumerate((r0_ref, r1_ref, r2_ref)):
        r = r_ref[0, :, :F] + r_ref[1, :, :F]
        ys.append(jax.nn.relu(r * dis[k][:, None] + b_ref[...]))
    cat = jnp.concatenate(ys, axis=1)
    h = lax.dot_general(cat, w_ref[...], (((1,), (1,)), ((), ())),
                        preferred_element_type=F32)
    z = jnp.zeros_like(h)
    o0_ref[...] = jnp.concatenate([h * dis[0][:, None], z], axis=1)
    o1_ref[...] = jnp.concatenate([h * dis[1][:, None], z], axis=1)
    o2_ref[...] = jnp.concatenate([h * dis[2][:, None], z], axis=1)


def _tc_final_body(r0_ref, r1_ref, r2_ref, deg_ref, b_ref, w_ref, cb_ref,
                   out_ref):
    dis = _dis_from_deg(deg_ref[...])
    ys = []
    for k, r_ref in enumerate((r0_ref, r1_ref, r2_ref)):
        r = r_ref[0, :, :F] + r_ref[1, :, :F]
        ys.append(jax.nn.relu(r * dis[k][:, None] + b_ref[...]))
    cat = jnp.concatenate(ys, axis=1)
    out = lax.dot_general(cat, w_ref[...], (((1,), (1,)), ((), ())),
                          preferred_element_type=F32)
    out_ref[...] = out + cb_ref[...]


def _tc1(x_pad, lin1_w, degp, n_pad):
    grid = (n_pad // BN,)
    return pl.pallas_call(
        _tc1_body,
        grid=grid,
        in_specs=[
            pl.BlockSpec((BN, 128), lambda i: (i, 0)),
            pl.BlockSpec((F, 128), lambda i: (0, 0)),
            pl.BlockSpec((NC, 3, BN), lambda i: (0, 0, i)),
        ],
        out_specs=[pl.BlockSpec((BN, 2 * F), lambda i: (i, 0))] * 3,
        out_shape=[jax.ShapeDtypeStruct((n_pad, 2 * F), F32)] * 3,
    )(x_pad, lin1_w, degp)


def _tc_mid(r0, r1, r2, degp, bias, w, n_pad):
    grid = (n_pad // BN,)
    return pl.pallas_call(
        _tc_mid_body,
        grid=grid,
        in_specs=[
            pl.BlockSpec((NC, BN, 2 * F), lambda i: (0, i, 0)),
            pl.BlockSpec((NC, BN, 2 * F), lambda i: (0, i, 0)),
            pl.BlockSpec((NC, BN, 2 * F), lambda i: (0, i, 0)),
            pl.BlockSpec((NC, 3, BN), lambda i: (0, 0, i)),
            pl.BlockSpec((1, F), lambda i: (0, 0)),
            pl.BlockSpec((F, 3 * F), lambda i: (0, 0)),
        ],
        out_specs=[pl.BlockSpec((BN, 2 * F), lambda i: (i, 0))] * 3,
        out_shape=[jax.ShapeDtypeStruct((n_pad, 2 * F), F32)] * 3,
    )(r0, r1, r2, degp, bias, w)


def _tc_final(r0, r1, r2, degp, bias, w, cb, n_pad, d_out):
    grid = (n_pad // BN,)
    return pl.pallas_call(
        _tc_final_body,
        grid=grid,
        in_specs=[
            pl.BlockSpec((NC, BN, 2 * F), lambda i: (0, i, 0)),
            pl.BlockSpec((NC, BN, 2 * F), lambda i: (0, i, 0)),
            pl.BlockSpec((NC, BN, 2 * F), lambda i: (0, i, 0)),
            pl.BlockSpec((NC, 3, BN), lambda i: (0, 0, i)),
            pl.BlockSpec((1, F), lambda i: (0, 0)),
            pl.BlockSpec((d_out, 3 * F), lambda i: (0, 0)),
            pl.BlockSpec((1, d_out), lambda i: (0, 0)),
        ],
        out_specs=pl.BlockSpec((BN, d_out), lambda i: (i, 0)),
        out_shape=jax.ShapeDtypeStruct((n_pad, d_out), F32),
    )(r0, r1, r2, degp, bias, w, cb)


def _prep_edges(src, dst, w, chunks, ch_slab, dummy):
    e_main = NW * chunks * CHUNK
    pad = e_main - src.shape[0]
    extra = ch_slab - chunks

    def prep_i(a):
        a = jnp.concatenate([a.astype(jnp.int32),
                             jnp.full((pad,), dummy, jnp.int32)])
        a = a.reshape(NW, chunks, CHUNK)
        return jnp.concatenate(
            [a, jnp.full((NW, extra, CHUNK), dummy, jnp.int32)], axis=1)

    wf = jnp.concatenate([w.astype(F32), jnp.zeros((pad,), F32)])
    wf = wf.reshape(NW, chunks, CHUNK)
    wf = jnp.concatenate([wf, jnp.zeros((NW, extra, CHUNK), F32)], axis=1)
    return prep_i(src), prep_i(dst), wf


def kernel(x, edge_index, edge_in, in_w, edge_out, out_w,
           lin1_w, bias1, linx0_w, biasx0, conv_w, conv_b):
    n, d_in = x.shape
    e = edge_index.shape[1]
    d_out = conv_w.shape[0]

    chunks = _cdiv(_cdiv(e, NW * CHUNK), NBUF) * NBUF
    ch_slab = chunks + 2
    n_pad = _cdiv(n + 1, NS * CHUNK) * NS * CHUNK
    rows_per_tile = n_pad // NS

    ones = jnp.ones((e,), F32)
    s0, d0, w0 = _prep_edges(edge_index[0], edge_index[1], ones, chunks, ch_slab, n)
    s1, d1, w1 = _prep_edges(edge_in[0], edge_in[1], in_w, chunks, ch_slab, n)
    s2, d2, w2 = _prep_edges(edge_out[0], edge_out[1], out_w, chunks, ch_slab, n)
    src_all = jnp.stack([s0, s1, s2])
    ew_all = jnp.stack([w0, w1, w2])

    x_pad = jnp.zeros((n_pad, d_in), F32).at[:n].set(x)
    zrow = jnp.zeros((rows_per_tile,), F32)
    zblk = jnp.zeros((CHUNK, 2 * F), F32)

    def expand(w):
        return jnp.broadcast_to(w.reshape(NW, ch_slab, CHUNK, 1),
                                (NW, ch_slab, CHUNK, LANES)).reshape(
                                    NW, ch_slab, LANES, CHUNK)

    w1x = expand(w1)
    w2x = expand(w2)

    deg_kernel = _make_deg_kernel(n_pad, ch_slab)
    conv_u = _make_conv_kernel(n_pad, ch_slab, ch_slab, weighted=False)
    conv_w_kernel = _make_conv_kernel(n_pad, ch_slab, ch_slab, weighted=True)

    degp = deg_kernel(src_all, ew_all, zrow).reshape(NC, 3, n_pad)

    h0, h1, h2 = _tc1(x_pad, lin1_w, degp, n_pad)

    dummy_ew = jnp.zeros((8, LANES, CHUNK), F32)

    r0 = conv_u(h0, s0, d0, dummy_ew, zblk)
    r1 = conv_w_kernel(h1, s1, d1, w1x, zblk)
    r2 = conv_w_kernel(h2, s2, d2, w2x, zblk)

    g0, g1, g2 = _tc_mid(r0, r1, r2, degp, bias1, linx0_w, n_pad)

    q0 = conv_u(g0, s0, d0, dummy_ew, zblk)
    q1 = conv_w_kernel(g1, s1, d1, w1x, zblk)
    q2 = conv_w_kernel(g2, s2, d2, w2x, zblk)

    out = _tc_final(q0, q1, q2, degp, biasx0, conv_w, conv_b.reshape(1, d_out),
                    n_pad, d_out)
    return out[:n]


# speedup vs baseline: 1.0453x; 1.0453x over previous
import functools
import math

import jax
import jax.numpy as jnp
from jax import lax
from jax.experimental import pallas as pl
from jax.experimental.pallas import tpu as pltpu
from jax.experimental.pallas import tpu_sc as plsc

F32 = jnp.float32

NC = 2
NS = 16
NW = NC * NS
LANES = 16

CHUNK = 128
F = 64
BN = 512


def _cdiv(a, b):
    return (a + b - 1) // b


def _make_deg_kernel(n_pad, chunks):
    rows_per_tile = n_pad // NS
    zsteps = rows_per_tile // CHUNK
    mesh = plsc.VectorSubcoreMesh(core_axis_name="c", subcore_axis_name="s")

    @functools.partial(
        pl.kernel,
        out_type=jax.ShapeDtypeStruct((NC * 3 * n_pad,), F32),
        mesh=mesh,
        scratch_types=[
            pltpu.VMEM((chunks, CHUNK), jnp.int32),
            pltpu.VMEM((chunks, CHUNK), F32),
            pltpu.VMEM((rows_per_tile,), F32),
            pltpu.VMEM_SHARED((n_pad,), F32),
            pltpu.VMEM_SHARED((n_pad,), F32),
            pltpu.VMEM_SHARED((n_pad,), F32),
        ],
    )
    def deg_kernel(src_hbm, ew_hbm, zrow_hbm, out_hbm,
                   src_v, ew_v, stage_v, acc0, acc1, acc2):
        c = lax.axis_index("c")
        s = lax.axis_index("s")
        wid = c * NS + s
        accs = [acc0, acc1, acc2]
        pltpu.sync_copy(zrow_hbm, stage_v)
        for acc in accs:
            pltpu.sync_copy(stage_v, acc.at[pl.ds(s * rows_per_tile, rows_per_tile)])
        plsc.subcore_barrier()
        for k, acc in enumerate(accs):
            pltpu.sync_copy(src_hbm.at[k, wid], src_v)
            pltpu.sync_copy(ew_hbm.at[k, wid], ew_v)

            def body(j, _, acc=acc):
                pltpu.sync_copy(ew_v.at[j], acc.at[src_v.at[j]], add=True)
                return 0

            lax.fori_loop(0, chunks, body, 0)
        plsc.subcore_barrier()
        for k, acc in enumerate(accs):
            pltpu.sync_copy(acc.at[pl.ds(s * rows_per_tile, rows_per_tile)], stage_v)
            off = (c * 3 + k) * n_pad + s * rows_per_tile
            pltpu.sync_copy(stage_v, out_hbm.at[pl.ds(off, rows_per_tile)])

    return deg_kernel


NBUF = 2


def _make_conv_kernel(n_pad, chunks, ch_slab, weighted):
    rows_per_tile = n_pad // NS
    zsteps = rows_per_tile // CHUNK
    W = 2 * F
    mesh = plsc.VectorSubcoreMesh(core_axis_name="c", subcore_axis_name="s")

    @functools.partial(
        pl.kernel,
        out_type=jax.ShapeDtypeStruct((NC, n_pad, W), F32),
        mesh=mesh,
        scratch_types=[
            pltpu.VMEM((ch_slab, CHUNK), jnp.int32),
            pltpu.VMEM((ch_slab, CHUNK), jnp.int32),
            pltpu.VMEM((CHUNK * LANES,), F32),
            pltpu.VMEM((CHUNK, W), F32),
            pltpu.VMEM_SHARED((n_pad, W), F32),
        ],
    )
    def conv_kernel(h_hbm, src_hbm, dst_hbm, ew_hbm, zblk_hbm, out_hbm,
                    src_v, dst_v, ew_v, rows_v, acc):
        c = lax.axis_index("c")
        s = lax.axis_index("s")
        wid = c * NS + s
        pltpu.sync_copy(zblk_hbm, rows_v)
        for z in range(zsteps):
            off = s * rows_per_tile + z * CHUNK
            pltpu.sync_copy(rows_v, acc.at[pl.ds(off, CHUNK)])
        plsc.subcore_barrier()
        pltpu.sync_copy(src_hbm.at[wid], src_v)
        pltpu.sync_copy(dst_hbm.at[wid], dst_v)

        def chunk_body(j, _):
            pltpu.sync_copy(h_hbm.at[src_v.at[j]], rows_v)
            if weighted:
                off = (wid * ch_slab + j) * (CHUNK * LANES)
                pltpu.sync_copy(ew_hbm.at[pl.ds(off, CHUNK * LANES)], ew_v)
                for i in range(CHUNK):
                    bv = ew_v[pl.ds(i * LANES, LANES)]
                    for q in range(F // LANES):
                        rows_v[i, pl.ds(q * LANES, LANES)] = (
                            rows_v[i, pl.ds(q * LANES, LANES)] * bv)
            pltpu.sync_copy(rows_v, acc.at[dst_v.at[j]], add=True)
            return 0

        lax.fori_loop(0, chunks, chunk_body, 0)
        plsc.subcore_barrier()
        for z in range(zsteps):
            off = s * rows_per_tile + z * CHUNK
            pltpu.sync_copy(acc.at[pl.ds(off, CHUNK)], rows_v)
            pltpu.sync_copy(rows_v, out_hbm.at[c, pl.ds(off, CHUNK)])

    return conv_kernel


def _make_block_kernel(n_pad, chunks, ch_slab):
    rows_per_tile = n_pad // NS
    zsteps = rows_per_tile // CHUNK
    W = 2 * F
    mesh = plsc.VectorSubcoreMesh(core_axis_name="c", subcore_axis_name="s")

    @functools.partial(
        pl.kernel,
        out_type=[jax.ShapeDtypeStruct((NC, n_pad, W), F32)] * 3,
        mesh=mesh,
        scratch_types=[
            pltpu.VMEM((NBUF, CHUNK), jnp.int32),
            pltpu.VMEM((NBUF, CHUNK), jnp.int32),
            pltpu.VMEM((NBUF, LANES, CHUNK), F32),
            pltpu.VMEM((NBUF, CHUNK, W), F32),
            pltpu.VMEM_SHARED((n_pad, W), F32),
        ] + [pltpu.SemaphoreType.DMA] * (4 * NBUF),
    )
    def block_kernel(h0_hbm, h1_hbm, h2_hbm, s0_hbm, s1_hbm, s2_hbm,
                     d0_hbm, d1_hbm, d2_hbm, ew1_hbm, ew2_hbm, zblk_hbm,
                     r0_hbm, r1_hbm, r2_hbm,
                     src_v, dst_v, ew_v, rows_v, acc, *sems):
        gsem = sems[0:NBUF]
        esem = sems[NBUF:2 * NBUF]
        isem = sems[2 * NBUF:3 * NBUF]
        dsem = sems[3 * NBUF:4 * NBUF]
        c = lax.axis_index("c")
        s = lax.axis_index("s")
        wid = c * NS + s

        def run_set(h_hbm, src_hbm, dst_hbm, ew_hbm, out_hbm, weighted):
            base = wid * (ch_slab * CHUNK)

            def idx_slice(j):
                return pl.ds(base + j * CHUNK, CHUNK)

            for b in range(NBUF):
                pltpu.sync_copy(zblk_hbm, rows_v.at[b])
            for z in range(zsteps):
                off = s * rows_per_tile + z * CHUNK
                pltpu.sync_copy(rows_v.at[0], acc.at[pl.ds(off, CHUNK)])
            plsc.subcore_barrier()
            pltpu.sync_copy(src_hbm.at[idx_slice(0)], src_v.at[0])
            pltpu.async_copy(dst_hbm.at[idx_slice(0)], dst_v.at[0], dsem[0])
            pltpu.async_copy(h_hbm.at[src_v.at[0]], rows_v.at[0], gsem[0])
            pltpu.async_copy(src_hbm.at[idx_slice(1)], src_v.at[1], isem[1])
            if weighted:
                pltpu.async_copy(ew_hbm.at[wid, 0], ew_v.at[0], esem[0])
                pltpu.async_copy(ew_hbm.at[wid, 1], ew_v.at[1], esem[1])

            def turn(j, b):
                o = 1 - b
                pltpu.async_copy(dst_hbm.at[idx_slice(j + 1)], dst_v.at[o],
                                 dsem[o])
                pltpu.make_async_copy(src_hbm.at[idx_slice(0)], src_v.at[o],
                                      isem[o]).wait()
                pltpu.async_copy(h_hbm.at[src_v.at[o]], rows_v.at[o],
                                 gsem[o])
                pltpu.make_async_copy(h_hbm.at[src_v.at[0]], rows_v.at[b],
                                      gsem[b]).wait()
                pltpu.async_copy(src_hbm.at[idx_slice(j + 2)], src_v.at[b],
                                 isem[b])
                if weighted:
                    pltpu.make_async_copy(ew_hbm.at[wid, 0], ew_v.at[b],
                                          esem[b]).wait()

                    def grp(g, _):
                        for l in range(LANES):
                            i = g * LANES + l
                            bv = ew_v[b, 2 * g + l // 8,
                                      pl.ds((l % 8) * LANES, LANES)]
                            for q in range(F // LANES):
                                rows_v[b, i, pl.ds(q * LANES, LANES)] = (
                                    rows_v[b, i, pl.ds(q * LANES, LANES)] * bv)
                        return 0

                    lax.fori_loop(0, CHUNK // LANES, grp, 0)
                    pltpu.async_copy(ew_hbm.at[wid, j + 2], ew_v.at[b],
                                     esem[b])
                pltpu.make_async_copy(dst_hbm.at[idx_slice(0)], dst_v.at[b],
                                      dsem[b]).wait()
                pltpu.sync_copy(rows_v.at[b], acc.at[dst_v.at[b]], add=True)

            def body(r, _):
                j0 = r * NBUF
                for b in range(NBUF):
                    turn(j0 + b, b)
                return 0

            lax.fori_loop(0, chunks // NBUF, body, 0)
            pltpu.make_async_copy(h_hbm.at[src_v.at[0]],
                                  rows_v.at[chunks % NBUF],
                                  gsem[chunks % NBUF]).wait()
            pltpu.make_async_copy(src_hbm.at[idx_slice(0)],
                                  src_v.at[(chunks + 1) % NBUF],
                                  isem[(chunks + 1) % NBUF]).wait()
            pltpu.make_async_copy(dst_hbm.at[idx_slice(0)],
                                  dst_v.at[chunks % NBUF],
                                  dsem[chunks % NBUF]).wait()
            if weighted:
                for b in range(NBUF):
                    pltpu.make_async_copy(ew_hbm.at[wid, 0], ew_v.at[b],
                                          esem[b]).wait()
            plsc.subcore_barrier()
            for z in range(zsteps):
                off = s * rows_per_tile + z * CHUNK
                pltpu.sync_copy(acc.at[pl.ds(off, CHUNK)], rows_v.at[0])
                pltpu.sync_copy(rows_v.at[0], out_hbm.at[c, pl.ds(off, CHUNK)])

        run_set(h0_hbm, s0_hbm, d0_hbm, None, r0_hbm, False)
        run_set(h1_hbm, s1_hbm, d1_hbm, ew1_hbm, r1_hbm, True)
        run_set(h2_hbm, s2_hbm, d2_hbm, ew2_hbm, r2_hbm, True)

    return block_kernel


def _dis_from_deg(deg_blk):
    d = deg_blk[0] + deg_blk[1]
    return jnp.where(d > 0, lax.rsqrt(d), jnp.zeros_like(d))


def _tc1_body(x_ref, w_ref, deg_ref, h0_ref, h1_ref, h2_ref):
    h = lax.dot_general(x_ref[...], w_ref[...], (((1,), (1,)), ((), ())),
                        preferred_element_type=F32)
    dis = _dis_from_deg(deg_ref[...])
    z = jnp.zeros_like(h)
    h0_ref[...] = jnp.concatenate([h * dis[0][:, None], z], axis=1)
    h1_ref[...] = jnp.concatenate([h * dis[1][:, None], z], axis=1)
    h2_ref[...] = jnp.concatenate([h * dis[2][:, None], z], axis=1)


def _tc_mid_body(r0_ref, r1_ref, r2_ref, deg_ref, b_ref, w_ref,
                 o0_ref, o1_ref, o2_ref):
    dis = _dis_from_deg(deg_ref[...])
    ys = []
    for k, r_ref in enumerate((r0_ref, r1_ref, r2_ref)):
        r = r_ref[0, :, :F] + r_ref[1, :, :F]
        ys.append(jax.nn.relu(r * dis[k][:, None] + b_ref[...]))
    cat = jnp.concatenate(ys, axis=1)
    h = lax.dot_general(cat, w_ref[...], (((1,), (1,)), ((), ())),
                        preferred_element_type=F32)
    z = jnp.zeros_like(h)
    o0_ref[...] = jnp.concatenate([h * dis[0][:, None], z], axis=1)
    o1_ref[...] = jnp.concatenate([h * dis[1][:, None], z], axis=1)
    o2_ref[...] = jnp.concatenate([h * dis[2][:, None], z], axis=1)


def _tc_final_body(r0_ref, r1_ref, r2_ref, deg_ref, b_ref, w_ref, cb_ref,
                   out_ref):
    dis = _dis_from_deg(deg_ref[...])
    ys = []
    for k, r_ref in enumerate((r0_ref, r1_ref, r2_ref)):
        r = r_ref[0, :, :F] + r_ref[1, :, :F]
        ys.append(jax.nn.relu(r * dis[k][:, None] + b_ref[...]))
    cat = jnp.concatenate(ys, axis=1)
    out = lax.dot_general(cat, w_ref[...], (((1,), (1,)), ((), ())),
                          preferred_element_type=F32)
    out_ref[...] = out + cb_ref[...]


def _tc1(x_pad, lin1_w, degp, n_pad):
    grid = (n_pad // BN,)
    return pl.pallas_call(
        _tc1_body,
        grid=grid,
        in_specs=[
            pl.BlockSpec((BN, 128), lambda i: (i, 0)),
            pl.BlockSpec((F, 128), lambda i: (0, 0)),
            pl.BlockSpec((NC, 3, BN), lambda i: (0, 0, i)),
        ],
        out_specs=[pl.BlockSpec((BN, 2 * F), lambda i: (i, 0))] * 3,
        out_shape=[jax.ShapeDtypeStruct((n_pad, 2 * F), F32)] * 3,
    )(x_pad, lin1_w, degp)


def _tc_mid(r0, r1, r2, degp, bias, w, n_pad):
    grid = (n_pad // BN,)
    return pl.pallas_call(
        _tc_mid_body,
        grid=grid,
        in_specs=[
            pl.BlockSpec((NC, BN, 2 * F), lambda i: (0, i, 0)),
            pl.BlockSpec((NC, BN, 2 * F), lambda i: (0, i, 0)),
            pl.BlockSpec((NC, BN, 2 * F), lambda i: (0, i, 0)),
            pl.BlockSpec((NC, 3, BN), lambda i: (0, 0, i)),
            pl.BlockSpec((1, F), lambda i: (0, 0)),
            pl.BlockSpec((F, 3 * F), lambda i: (0, 0)),
        ],
        out_specs=[pl.BlockSpec((BN, 2 * F), lambda i: (i, 0))] * 3,
        out_shape=[jax.ShapeDtypeStruct((n_pad, 2 * F), F32)] * 3,
    )(r0, r1, r2, degp, bias, w)


def _tc_final(r0, r1, r2, degp, bias, w, cb, n_pad, d_out):
    grid = (n_pad // BN,)
    return pl.pallas_call(
        _tc_final_body,
        grid=grid,
        in_specs=[
            pl.BlockSpec((NC, BN, 2 * F), lambda i: (0, i, 0)),
            pl.BlockSpec((NC, BN, 2 * F), lambda i: (0, i, 0)),
            pl.BlockSpec((NC, BN, 2 * F), lambda i: (0, i, 0)),
            pl.BlockSpec((NC, 3, BN), lambda i: (0, 0, i)),
            pl.BlockSpec((1, F), lambda i: (0, 0)),
            pl.BlockSpec((d_out, 3 * F), lambda i: (0, 0)),
            pl.BlockSpec((1, d_out), lambda i: (0, 0)),
        ],
        out_specs=pl.BlockSpec((BN, d_out), lambda i: (i, 0)),
        out_shape=jax.ShapeDtypeStruct((n_pad, d_out), F32),
    )(r0, r1, r2, degp, bias, w, cb)


def _prep_edges(src, dst, w, chunks, ch_slab, dummy):
    e_main = NW * chunks * CHUNK
    pad = e_main - src.shape[0]
    extra = ch_slab - chunks

    def prep_i(a):
        a = jnp.concatenate([a.astype(jnp.int32),
                             jnp.full((pad,), dummy, jnp.int32)])
        a = a.reshape(NW, chunks, CHUNK)
        return jnp.concatenate(
            [a, jnp.full((NW, extra, CHUNK), dummy, jnp.int32)], axis=1)

    wf = jnp.concatenate([w.astype(F32), jnp.zeros((pad,), F32)])
    wf = wf.reshape(NW, chunks, CHUNK)
    wf = jnp.concatenate([wf, jnp.zeros((NW, extra, CHUNK), F32)], axis=1)
    return prep_i(src), prep_i(dst), wf


def kernel(x, edge_index, edge_in, in_w, edge_out, out_w,
           lin1_w, bias1, linx0_w, biasx0, conv_w, conv_b):
    n, d_in = x.shape
    e = edge_index.shape[1]
    d_out = conv_w.shape[0]

    chunks = _cdiv(_cdiv(e, NW * CHUNK), NBUF) * NBUF
    ch_slab = chunks + 2
    n_pad = _cdiv(n + 1, NS * CHUNK) * NS * CHUNK
    rows_per_tile = n_pad // NS

    ones = jnp.ones((e,), F32)
    s0, d0, w0 = _prep_edges(edge_index[0], edge_index[1], ones, chunks, ch_slab, n)
    s1, d1, w1 = _prep_edges(edge_in[0], edge_in[1], in_w, chunks, ch_slab, n)
    s2, d2, w2 = _prep_edges(edge_out[0], edge_out[1], out_w, chunks, ch_slab, n)
    src_all = jnp.stack([s0, s1, s2])
    ew_all = jnp.stack([w0, w1, w2])

    x_pad = jnp.zeros((n_pad, d_in), F32).at[:n].set(x)
    zrow = jnp.zeros((rows_per_tile,), F32)
    zblk = jnp.zeros((CHUNK, 2 * F), F32)

    def expand(w):
        return jnp.broadcast_to(w.reshape(NW, ch_slab, CHUNK, 1),
                                (NW, ch_slab, CHUNK, LANES)).reshape(-1)

    w1x = expand(w1)
    w2x = expand(w2)

    deg_kernel = _make_deg_kernel(n_pad, ch_slab)
    conv_u = _make_conv_kernel(n_pad, ch_slab, ch_slab, weighted=False)
    conv_w_kernel = _make_conv_kernel(n_pad, ch_slab, ch_slab, weighted=True)

    degp = deg_kernel(src_all, ew_all, zrow).reshape(NC, 3, n_pad)

    h0, h1, h2 = _tc1(x_pad, lin1_w, degp, n_pad)

    dummy_ew = jnp.zeros((8,), F32)

    r0 = conv_u(h0, s0, d0, dummy_ew, zblk)
    r1 = conv_w_kernel(h1, s1, d1, w1x, zblk)
    r2 = conv_w_kernel(h2, s2, d2, w2x, zblk)

    g0, g1, g2 = _tc_mid(r0, r1, r2, degp, bias1, linx0_w, n_pad)

    q0 = conv_u(g0, s0, d0, dummy_ew, zblk)
    q1 = conv_w_kernel(g1, s1, d1, w1x, zblk)
    q2 = conv_w_kernel(g2, s2, d2, w2x, zblk)

    out = _tc_final(q0, q1, q2, degp, biasx0, conv_w, conv_b.reshape(1, d_out),
                    n_pad, d_out)
    return out[:n]


# speedup vs baseline: 2.1124x; 2.0209x over previous
import functools
import math

import jax
import jax.numpy as jnp
from jax import lax
from jax.experimental import pallas as pl
from jax.experimental.pallas import tpu as pltpu
from jax.experimental.pallas import tpu_sc as plsc

F32 = jnp.float32

NC = 2
NS = 16
NW = NC * NS
LANES = 16

CHUNK = 128
F = 64
BN = 512


def _cdiv(a, b):
    return (a + b - 1) // b


def _make_deg_kernel(n_pad, chunks):
    rows_per_tile = n_pad // NS
    zsteps = rows_per_tile // CHUNK
    mesh = plsc.VectorSubcoreMesh(core_axis_name="c", subcore_axis_name="s")

    @functools.partial(
        pl.kernel,
        out_type=jax.ShapeDtypeStruct((NC * 3 * n_pad,), F32),
        mesh=mesh,
        scratch_types=[
            pltpu.VMEM((chunks, CHUNK), jnp.int32),
            pltpu.VMEM((chunks, CHUNK), F32),
            pltpu.VMEM((rows_per_tile,), F32),
            pltpu.VMEM_SHARED((n_pad,), F32),
            pltpu.VMEM_SHARED((n_pad,), F32),
            pltpu.VMEM_SHARED((n_pad,), F32),
        ],
    )
    def deg_kernel(src_hbm, ew_hbm, zrow_hbm, out_hbm,
                   src_v, ew_v, stage_v, acc0, acc1, acc2):
        c = lax.axis_index("c")
        s = lax.axis_index("s")
        wid = c * NS + s
        accs = [acc0, acc1, acc2]
        pltpu.sync_copy(zrow_hbm, stage_v)
        for acc in accs:
            pltpu.sync_copy(stage_v, acc.at[pl.ds(s * rows_per_tile, rows_per_tile)])
        plsc.subcore_barrier()
        for k, acc in enumerate(accs):
            pltpu.sync_copy(src_hbm.at[k, wid], src_v)
            pltpu.sync_copy(ew_hbm.at[k, wid], ew_v)

            def body(j, _, acc=acc):
                pltpu.sync_copy(ew_v.at[j], acc.at[src_v.at[j]], add=True)
                return 0

            lax.fori_loop(0, chunks, body, 0)
        plsc.subcore_barrier()
        for k, acc in enumerate(accs):
            pltpu.sync_copy(acc.at[pl.ds(s * rows_per_tile, rows_per_tile)], stage_v)
            off = (c * 3 + k) * n_pad + s * rows_per_tile
            pltpu.sync_copy(stage_v, out_hbm.at[pl.ds(off, rows_per_tile)])

    return deg_kernel


NBUF = 2


def _make_conv_kernel(n_pad, chunks, ch_slab, weighted):
    rows_per_tile = n_pad // NS
    zsteps = rows_per_tile // CHUNK
    W = 2 * F
    mesh = plsc.VectorSubcoreMesh(core_axis_name="c", subcore_axis_name="s")

    @functools.partial(
        pl.kernel,
        out_type=jax.ShapeDtypeStruct((NC, n_pad, W), F32),
        mesh=mesh,
        scratch_types=[
            pltpu.VMEM((ch_slab, CHUNK), jnp.int32),
            pltpu.VMEM((ch_slab, CHUNK), jnp.int32),
            pltpu.VMEM((CHUNK * LANES,), F32),
            pltpu.VMEM((CHUNK, W), F32),
            pltpu.VMEM_SHARED((n_pad, W), F32),
        ],
    )
    def conv_kernel(h_hbm, src_hbm, dst_hbm, ew_hbm, zblk_hbm, out_hbm,
                    src_v, dst_v, ew_v, rows_v, acc):
        c = lax.axis_index("c")
        s = lax.axis_index("s")
        wid = c * NS + s
        pltpu.sync_copy(zblk_hbm, rows_v)
        for z in range(zsteps):
            off = s * rows_per_tile + z * CHUNK
            pltpu.sync_copy(rows_v, acc.at[pl.ds(off, CHUNK)])
        plsc.subcore_barrier()
        pltpu.sync_copy(src_hbm.at[wid], src_v)
        pltpu.sync_copy(dst_hbm.at[wid], dst_v)

        def chunk_body(j, _):
            pltpu.sync_copy(h_hbm.at[src_v.at[j]], rows_v)
            if weighted:
                off = (wid * ch_slab + j) * (CHUNK * LANES)
                pltpu.sync_copy(ew_hbm.at[pl.ds(off, CHUNK * LANES)], ew_v)
                for i in range(CHUNK):
                    bv = ew_v[pl.ds(i * LANES, LANES)]
                    for q in range(F // LANES):
                        rows_v[i, pl.ds(q * LANES, LANES)] = (
                            rows_v[i, pl.ds(q * LANES, LANES)] * bv)
            pltpu.sync_copy(rows_v, acc.at[dst_v.at[j]], add=True)
            return 0

        lax.fori_loop(0, chunks, chunk_body, 0)
        plsc.subcore_barrier()
        for z in range(zsteps):
            off = s * rows_per_tile + z * CHUNK
            pltpu.sync_copy(acc.at[pl.ds(off, CHUNK)], rows_v)
            pltpu.sync_copy(rows_v, out_hbm.at[c, pl.ds(off, CHUNK)])

    return conv_kernel


def _make_block_kernel(n_pad, chunks, ch_slab):
    rows_per_tile = n_pad // NS
    zsteps = rows_per_tile // CHUNK
    W = 2 * F
    mesh = plsc.VectorSubcoreMesh(core_axis_name="c", subcore_axis_name="s")

    @functools.partial(
        pl.kernel,
        out_type=[jax.ShapeDtypeStruct((NC, n_pad, W), F32)] * 3,
        mesh=mesh,
        scratch_types=[
            pltpu.VMEM((NBUF, CHUNK), jnp.int32),
            pltpu.VMEM((NBUF, CHUNK), jnp.int32),
            pltpu.VMEM((NBUF, LANES, CHUNK), F32),
            pltpu.VMEM((NBUF, CHUNK, W), F32),
            pltpu.VMEM_SHARED((n_pad, W), F32),
        ] + [pltpu.SemaphoreType.DMA] * (4 * NBUF),
    )
    def block_kernel(h0_hbm, h1_hbm, h2_hbm, s0_hbm, s1_hbm, s2_hbm,
                     d0_hbm, d1_hbm, d2_hbm, ew1_hbm, ew2_hbm, zblk_hbm,
                     r0_hbm, r1_hbm, r2_hbm,
                     src_v, dst_v, ew_v, rows_v, acc, *sems):
        gsem = sems[0:NBUF]
        esem = sems[NBUF:2 * NBUF]
        isem = sems[2 * NBUF:3 * NBUF]
        dsem = sems[3 * NBUF:4 * NBUF]
        c = lax.axis_index("c")
        s = lax.axis_index("s")
        wid = c * NS + s

        def run_set(h_hbm, src_hbm, dst_hbm, ew_hbm, out_hbm, weighted):
            base = wid * (ch_slab * CHUNK)

            def idx_slice(j):
                return pl.ds(base + j * CHUNK, CHUNK)

            for b in range(NBUF):
                pltpu.sync_copy(zblk_hbm, rows_v.at[b])
            for z in range(zsteps):
                off = s * rows_per_tile + z * CHUNK
                pltpu.sync_copy(rows_v.at[0], acc.at[pl.ds(off, CHUNK)])
            plsc.subcore_barrier()
            pltpu.sync_copy(src_hbm.at[idx_slice(0)], src_v.at[0])
            pltpu.async_copy(dst_hbm.at[idx_slice(0)], dst_v.at[0], dsem[0])
            pltpu.async_copy(h_hbm.at[src_v.at[0]], rows_v.at[0], gsem[0])
            pltpu.async_copy(src_hbm.at[idx_slice(1)], src_v.at[1], isem[1])
            if weighted:
                pltpu.async_copy(ew_hbm.at[wid, 0], ew_v.at[0], esem[0])
                pltpu.async_copy(ew_hbm.at[wid, 1], ew_v.at[1], esem[1])

            def turn(j, b):
                o = 1 - b
                pltpu.async_copy(dst_hbm.at[idx_slice(j + 1)], dst_v.at[o],
                                 dsem[o])
                pltpu.make_async_copy(src_hbm.at[idx_slice(0)], src_v.at[o],
                                      isem[o]).wait()
                pltpu.async_copy(h_hbm.at[src_v.at[o]], rows_v.at[o],
                                 gsem[o])
                pltpu.make_async_copy(h_hbm.at[src_v.at[0]], rows_v.at[b],
                                      gsem[b]).wait()
                pltpu.async_copy(src_hbm.at[idx_slice(j + 2)], src_v.at[b],
                                 isem[b])
                if weighted:
                    pltpu.make_async_copy(ew_hbm.at[wid, 0], ew_v.at[b],
                                          esem[b]).wait()

                    def grp(g, _):
                        for l in range(LANES):
                            i = g * LANES + l
                            bv = ew_v[b, 2 * g + l // 8,
                                      pl.ds((l % 8) * LANES, LANES)]
                            for q in range(F // LANES):
                                rows_v[b, i, pl.ds(q * LANES, LANES)] = (
                                    rows_v[b, i, pl.ds(q * LANES, LANES)] * bv)
                        return 0

                    lax.fori_loop(0, CHUNK // LANES, grp, 0)
                    pltpu.async_copy(ew_hbm.at[wid, j + 2], ew_v.at[b],
                                     esem[b])
                pltpu.make_async_copy(dst_hbm.at[idx_slice(0)], dst_v.at[b],
                                      dsem[b]).wait()
                pltpu.sync_copy(rows_v.at[b], acc.at[dst_v.at[b]], add=True)

            def body(r, _):
                j0 = r * NBUF
                for b in range(NBUF):
                    turn(j0 + b, b)
                return 0

            lax.fori_loop(0, chunks // NBUF, body, 0)
            pltpu.make_async_copy(h_hbm.at[src_v.at[0]],
                                  rows_v.at[chunks % NBUF],
                                  gsem[chunks % NBUF]).wait()
            pltpu.make_async_copy(src_hbm.at[idx_slice(0)],
                                  src_v.at[(chunks + 1) % NBUF],
                                  isem[(chunks + 1) % NBUF]).wait()
            pltpu.make_async_copy(dst_hbm.at[idx_slice(0)],
                                  dst_v.at[chunks % NBUF],
                                  dsem[chunks % NBUF]).wait()
            if weighted:
                for b in range(NBUF):
                    pltpu.make_async_copy(ew_hbm.at[wid, 0], ew_v.at[b],
                                          esem[b]).wait()
            plsc.subcore_barrier()
            for z in range(zsteps):
                off = s * rows_per_tile + z * CHUNK
                pltpu.sync_copy(acc.at[pl.ds(off, CHUNK)], rows_v.at[0])
                pltpu.sync_copy(rows_v.at[0], out_hbm.at[c, pl.ds(off, CHUNK)])

        run_set(h0_hbm, s0_hbm, d0_hbm, None, r0_hbm, False)
        run_set(h1_hbm, s1_hbm, d1_hbm, ew1_hbm, r1_hbm, True)
        run_set(h2_hbm, s2_hbm, d2_hbm, ew2_hbm, r2_hbm, True)

    return block_kernel


def _dis_from_deg(deg_blk):
    d = deg_blk[0] + deg_blk[1]
    return jnp.where(d > 0, lax.rsqrt(d), jnp.zeros_like(d))


def _tc1_body(x_ref, w_ref, deg_ref, h0_ref, h1_ref, h2_ref):
    h = lax.dot_general(x_ref[...], w_ref[...], (((1,), (1,)), ((), ())),
                        preferred_element_type=F32)
    dis = _dis_from_deg(deg_ref[...])
    z = jnp.zeros_like(h)
    h0_ref[...] = jnp.concatenate([h * dis[0][:, None], z], axis=1)
    h1_ref[...] = jnp.concatenate([h * dis[1][:, None], z], axis=1)
    h2_ref[...] = jnp.concatenate([h * dis[2][:, None], z], axis=1)


def _tc_mid_body(r0_ref, r1_ref, r2_ref, deg_ref, b_ref, w_ref,
                 o0_ref, o1_ref, o2_ref):
    dis = _dis_from_deg(deg_ref[...])
    ys = []
    for k, r_ref in enumerate((r0_ref, r1_ref, r2_ref)):
        r = r_ref[0, :, :F] + r_ref[1, :, :F]
        ys.append(jax.nn.relu(r * dis[k][:, None] + b_ref[...]))
    cat = jnp.concatenate(ys, axis=1)
    h = lax.dot_general(cat, w_ref[...], (((1,), (1,)), ((), ())),
                        preferred_element_type=F32)
    z = jnp.zeros_like(h)
    o0_ref[...] = jnp.concatenate([h * dis[0][:, None], z], axis=1)
    o1_ref[...] = jnp.concatenate([h * dis[1][:, None], z], axis=1)
    o2_ref[...] = jnp.concatenate([h * dis[2][:, None], z], axis=1)


def _tc_final_body(r0_ref, r1_ref, r2_ref, deg_ref, b_ref, w_ref, cb_ref,
                   out_ref):
    dis = _dis_from_deg(deg_ref[...])
    ys = []
    for k, r_ref in enumerate((r0_ref, r1_ref, r2_ref)):
        r = r_ref[0, :, :F] + r_ref[1, :, :F]
        ys.append(jax.nn.relu(r * dis[k][:, None] + b_ref[...]))
    cat = jnp.concatenate(ys, axis=1)
    out = lax.dot_general(cat, w_ref[...], (((1,), (1,)), ((), ())),
                          preferred_element_type=F32)
    out_ref[...] = out + cb_ref[...]


def _tc1(x_pad, lin1_w, degp, n_pad):
    grid = (n_pad // BN,)
    return pl.pallas_call(
        _tc1_body,
        grid=grid,
        in_specs=[
            pl.BlockSpec((BN, 128), lambda i: (i, 0)),
            pl.BlockSpec((F, 128), lambda i: (0, 0)),
            pl.BlockSpec((NC, 3, BN), lambda i: (0, 0, i)),
        ],
        out_specs=[pl.BlockSpec((BN, 2 * F), lambda i: (i, 0))] * 3,
        out_shape=[jax.ShapeDtypeStruct((n_pad, 2 * F), F32)] * 3,
    )(x_pad, lin1_w, degp)


def _tc_mid(r0, r1, r2, degp, bias, w, n_pad):
    grid = (n_pad // BN,)
    return pl.pallas_call(
        _tc_mid_body,
        grid=grid,
        in_specs=[
            pl.BlockSpec((NC, BN, 2 * F), lambda i: (0, i, 0)),
            pl.BlockSpec((NC, BN, 2 * F), lambda i: (0, i, 0)),
            pl.BlockSpec((NC, BN, 2 * F), lambda i: (0, i, 0)),
            pl.BlockSpec((NC, 3, BN), lambda i: (0, 0, i)),
            pl.BlockSpec((1, F), lambda i: (0, 0)),
            pl.BlockSpec((F, 3 * F), lambda i: (0, 0)),
        ],
        out_specs=[pl.BlockSpec((BN, 2 * F), lambda i: (i, 0))] * 3,
        out_shape=[jax.ShapeDtypeStruct((n_pad, 2 * F), F32)] * 3,
    )(r0, r1, r2, degp, bias, w)


def _tc_final(r0, r1, r2, degp, bias, w, cb, n_pad, d_out):
    grid = (n_pad // BN,)
    return pl.pallas_call(
        _tc_final_body,
        grid=grid,
        in_specs=[
            pl.BlockSpec((NC, BN, 2 * F), lambda i: (0, i, 0)),
            pl.BlockSpec((NC, BN, 2 * F), lambda i: (0, i, 0)),
            pl.BlockSpec((NC, BN, 2 * F), lambda i: (0, i, 0)),
            pl.BlockSpec((NC, 3, BN), lambda i: (0, 0, i)),
            pl.BlockSpec((1, F), lambda i: (0, 0)),
            pl.BlockSpec((d_out, 3 * F), lambda i: (0, 0)),
            pl.BlockSpec((1, d_out), lambda i: (0, 0)),
        ],
        out_specs=pl.BlockSpec((BN, d_out), lambda i: (i, 0)),
        out_shape=jax.ShapeDtypeStruct((n_pad, d_out), F32),
    )(r0, r1, r2, degp, bias, w, cb)


def _prep_edges(src, dst, w, chunks, dummy):
    e_main = NW * chunks * CHUNK
    pad = e_main - src.shape[0]

    def prep_i(a):
        a = jnp.concatenate([a.astype(jnp.int32),
                             jnp.full((pad,), dummy, jnp.int32)])
        return a.reshape(NW, chunks, CHUNK)

    wf = jnp.concatenate([w.astype(F32), jnp.zeros((pad,), F32)])
    return prep_i(src), prep_i(dst), wf.reshape(NW, chunks, CHUNK)


def kernel(x, edge_index, edge_in, in_w, edge_out, out_w,
           lin1_w, bias1, linx0_w, biasx0, conv_w, conv_b):
    n, d_in = x.shape
    e = edge_index.shape[1]
    d_out = conv_w.shape[0]

    chunks = _cdiv(e, NW * CHUNK)
    ch_slab = chunks
    n_pad = _cdiv(n + 1, NS * CHUNK) * NS * CHUNK
    rows_per_tile = n_pad // NS

    ones = jnp.ones((e,), F32)
    s0, d0, w0 = _prep_edges(edge_index[0], edge_index[1], ones, chunks, n)
    s1, d1, w1 = _prep_edges(edge_in[0], edge_in[1], in_w, chunks, n)
    s2, d2, w2 = _prep_edges(edge_out[0], edge_out[1], out_w, chunks, n)
    src_all = jnp.stack([s0, s1, s2])
    ew_all = jnp.stack([w0, w1, w2])

    x_pad = jnp.zeros((n_pad, d_in), F32).at[:n].set(x)
    zrow = jnp.zeros((rows_per_tile,), F32)
    zblk = jnp.zeros((CHUNK, 2 * F), F32)

    def expand(w):
        return jnp.broadcast_to(w.reshape(NW, ch_slab, CHUNK, 1),
                                (NW, ch_slab, CHUNK, LANES)).reshape(-1)

    w1x = expand(w1)
    w2x = expand(w2)

    deg_kernel = _make_deg_kernel(n_pad, ch_slab)
    conv_u = _make_conv_kernel(n_pad, ch_slab, ch_slab, weighted=False)
    conv_w_kernel = _make_conv_kernel(n_pad, ch_slab, ch_slab, weighted=True)

    degp = deg_kernel(src_all, ew_all, zrow).reshape(NC, 3, n_pad)

    h0, h1, h2 = _tc1(x_pad, lin1_w, degp, n_pad)

    dummy_ew = jnp.zeros((8,), F32)

    r0 = conv_u(h0, s0, d0, dummy_ew, zblk)
    r1 = conv_w_kernel(h1, s1, d1, w1x, zblk)
    r2 = conv_w_kernel(h2, s2, d2, w2x, zblk)

    g0, g1, g2 = _tc_mid(r0, r1, r2, degp, bias1, linx0_w, n_pad)

    q0 = conv_u(g0, s0, d0, dummy_ew, zblk)
    q1 = conv_w_kernel(g1, s1, d1, w1x, zblk)
    q2 = conv_w_kernel(g2, s2, d2, w2x, zblk)

    out = _tc_final(q0, q1, q2, degp, biasx0, conv_w, conv_b.reshape(1, d_out),
                    n_pad, d_out)
    return out[:n]
